# Initial kernel scaffold; baseline (speedup 1.0000x reference)
import jax
import jax.numpy as jnp
from jax.experimental import pallas as pl


def kernel(pos, edge_index, W1, b1, W2, b2, W3, b3, W4, b4, W5, b5, fcW, fcb):
    raise NotImplementedError("write your pallas kernel here")


# speedup vs baseline: 4.9524x; 4.9524x over previous
import functools

import jax
import jax.numpy as jnp
from jax import lax
from jax.experimental import pallas as pl
from jax.experimental.pallas import tpu as pltpu
from jax.experimental.pallas import tpu_sc as plsc

N = 10000
E = 160000
NPAD = 10240
NC, NS = 2, 16
NW = NC * NS
B = 128
EPW = 5120
NB = EPW // B
EPAD = NW * EPW
RPT = NPAD // NS
RB = 1000
DC = 128


def _make_sc_scatter(nch, gather=True):
  mesh = plsc.VectorSubcoreMesh(core_axis_name="c", subcore_axis_name="s")

  @functools.partial(
      pl.kernel,
      mesh=mesh,
      out_type=jax.ShapeDtypeStruct((NC, nch, NPAD, DC), jnp.float32),
      scratch_types=[
          pltpu.VMEM((NB, B), jnp.int32),
          pltpu.VMEM((NB, B), jnp.int32),
          pltpu.VMEM((B, DC), jnp.float32),
          pltpu.VMEM_SHARED((NPAD, DC), jnp.float32),
      ],
  )
  def sc_kernel(u_hbm, src_hbm, dst_hbm, zeros_hbm, out_hbm,
                src_v, dst_v, rows_v, acc):
    c = lax.axis_index("c")
    s = lax.axis_index("s")
    wid = c * NS + s
    pltpu.sync_copy(src_hbm.at[wid], src_v)
    pltpu.sync_copy(dst_hbm.at[wid], dst_v)
    if not gather:
      pltpu.sync_copy(u_hbm, rows_v)
    for ch in range(nch):
      pltpu.sync_copy(zeros_hbm, acc.at[pl.ds(s * RPT, RPT)])
      plsc.subcore_barrier()

      def body(j, carry):
        if gather:
          pltpu.sync_copy(u_hbm.at[ch].at[src_v.at[j]], rows_v)
        pltpu.sync_copy(rows_v, acc.at[dst_v.at[j]], add=True)
        return carry

      lax.fori_loop(0, NB, body, 0)
      plsc.subcore_barrier()
      pltpu.sync_copy(acc.at[pl.ds(s * RPT, RPT)],
                      out_hbm.at[c, ch].at[pl.ds(s * RPT, RPT)])

  return sc_kernel


def _tc_prep_kernel(degp_ref, pos_ref, dinv_ref, u0_ref):
  deg = degp_ref[0, 0, :, 0:1] + degp_ref[1, 0, :, 0:1] + 1.0
  dinv = lax.rsqrt(deg)
  dinv_ref[...] = jnp.broadcast_to(dinv, (RB, 128))
  u0_ref[0] = dinv * pos_ref[...]


def _tc_prep(degp, pos_pad):
  return pl.pallas_call(
      _tc_prep_kernel,
      grid=(N // RB,),
      in_specs=[
          pl.BlockSpec((NC, 1, RB, DC), lambda i: (0, 0, i, 0)),
          pl.BlockSpec((RB, DC), lambda i: (i, 0)),
      ],
      out_specs=[
          pl.BlockSpec((RB, 128), lambda i: (i, 0)),
          pl.BlockSpec((1, RB, DC), lambda i: (0, i, 0)),
      ],
      out_shape=[
          jax.ShapeDtypeStruct((NPAD, 128), jnp.float32),
          jax.ShapeDtypeStruct((1, NPAD, DC), jnp.float32),
      ],
  )(degp, pos_pad)


def _tc_layer_kernel(nch_in, nch_out, d_out,
                     p_ref, u_ref, dinv_ref, w_ref, b_ref, un_ref):
  parts = [p_ref[0, ch] + p_ref[1, ch] + u_ref[ch] for ch in range(nch_in)]
  agg = parts[0] if nch_in == 1 else jnp.concatenate(parts, axis=-1)
  dv = dinv_ref[:, 0:1]
  s = dv * agg
  y = jnp.dot(s, w_ref[...], preferred_element_type=jnp.float32) + b_ref[...]
  x = jnp.where(y >= 0.0, y, 0.01 * y)
  un = dv * x
  if d_out < nch_out * DC:
    un = jnp.concatenate(
        [un, jnp.zeros((RB, nch_out * DC - d_out), jnp.float32)], axis=-1)
  for ch in range(nch_out):
    un_ref[ch] = un[:, ch * DC:(ch + 1) * DC]


def _tc_layer(p, u, dinvb, w, b, nch_in, nch_out):
  d_in, d_out = w.shape
  return pl.pallas_call(
      functools.partial(_tc_layer_kernel, nch_in, nch_out, d_out),
      grid=(N // RB,),
      in_specs=[
          pl.BlockSpec((NC, nch_in, RB, DC), lambda i: (0, 0, i, 0)),
          pl.BlockSpec((nch_in, RB, DC), lambda i: (0, i, 0)),
          pl.BlockSpec((RB, 128), lambda i: (i, 0)),
          pl.BlockSpec((d_in, d_out), lambda i: (0, 0)),
          pl.BlockSpec((1, d_out), lambda i: (0, 0)),
      ],
      out_specs=pl.BlockSpec((nch_out, RB, DC), lambda i: (0, i, 0)),
      out_shape=jax.ShapeDtypeStruct((nch_out, NPAD, DC), jnp.float32),
  )(p, u, dinvb, w, b)


def _tc_final_kernel(nch_in,
                     p_ref, u_ref, dinv_ref, w_ref, b_ref, fcw_ref, fcb_ref,
                     out_ref, macc):
  i = pl.program_id(0)
  parts = [p_ref[0, ch] + p_ref[1, ch] + u_ref[ch] for ch in range(nch_in)]
  agg = jnp.concatenate(parts, axis=-1)
  dv = dinv_ref[:, 0:1]
  s = dv * agg
  y = jnp.dot(s, w_ref[...], preferred_element_type=jnp.float32) + b_ref[...]
  x = jnp.where(y >= 0.0, y, 0.01 * y)
  blk_max = jnp.max(x, axis=0, keepdims=True)

  @pl.when(i == 0)
  def _():
    macc[...] = blk_max

  @pl.when(i > 0)
  def _():
    macc[...] = jnp.maximum(macc[...], blk_max)

  @pl.when(i == pl.num_programs(0) - 1)
  def _():
    r = macc[...]
    out = lax.dot_general(r, fcw_ref[...], (((1,), (1,)), ((), ())),
                          preferred_element_type=jnp.float32)
    out_ref[...] = out + fcb_ref[...]


def _tc_final(p, u, dinvb, w, b, fcw, fcb, nch_in):
  d_in, d_out = w.shape
  return pl.pallas_call(
      functools.partial(_tc_final_kernel, nch_in),
      grid=(N // RB,),
      in_specs=[
          pl.BlockSpec((NC, nch_in, RB, DC), lambda i: (0, 0, i, 0)),
          pl.BlockSpec((nch_in, RB, DC), lambda i: (0, i, 0)),
          pl.BlockSpec((RB, 128), lambda i: (i, 0)),
          pl.BlockSpec((d_in, d_out), lambda i: (0, 0)),
          pl.BlockSpec((1, d_out), lambda i: (0, 0)),
          pl.BlockSpec((1024, 1024), lambda i: (0, 0)),
          pl.BlockSpec((1, 1024), lambda i: (0, 0)),
      ],
      out_specs=pl.BlockSpec((1, 1024), lambda i: (0, 0)),
      out_shape=jax.ShapeDtypeStruct((1, 1024), jnp.float32),
      scratch_shapes=[pltpu.VMEM((1, 1024), jnp.float32)],
  )(p, u, dinvb, w, b, fcw, fcb)


def kernel(pos, edge_index, W1, b1, W2, b2, W3, b3, W4, b4, W5, b5, fcW, fcb):
  f32 = jnp.float32
  src = edge_index[0]
  dst = edge_index[1]
  pad = EPAD - E
  src_r = jnp.concatenate([src, jnp.zeros((pad,), jnp.int32)]).reshape(NW, NB, B)
  dst_r = jnp.concatenate([dst, jnp.full((pad,), N, jnp.int32)]).reshape(NW, NB, B)

  pos_pad = jnp.zeros((NPAD, DC), f32).at[:N, :3].set(pos)
  w1_pad = jnp.zeros((DC, 64), f32).at[:3].set(W1)
  w2_pad = jnp.zeros((DC, 128), f32).at[:64].set(W2)
  zeros_rpt = jnp.zeros((RPT, DC), f32)
  ones_b = jnp.ones((B, DC), f32)

  degp = _make_sc_scatter(1, gather=False)(ones_b, src_r, dst_r, zeros_rpt)

  dinvb, u = _tc_prep(degp, pos_pad)

  layer_ws = [(w1_pad, b1), (w2_pad, b2), (W3, b3), (W4, b4), (W5, b5)]
  nchs = [1, 1, 1, 2, 4]

  for l in range(4):
    w, b = layer_ws[l]
    p = _make_sc_scatter(nchs[l])(u, src_r, dst_r, zeros_rpt)
    u = _tc_layer(p, u, dinvb, w, b.reshape(1, -1), nchs[l], nchs[l + 1])

  w5, b5_ = layer_ws[4]
  p = _make_sc_scatter(nchs[4])(u, src_r, dst_r, zeros_rpt)
  out = _tc_final(p, u, dinvb, w5, b5_.reshape(1, -1), fcW, fcb.reshape(1, -1),
                  nchs[4])
  return out.reshape(-1)


# speedup vs baseline: 5.3992x; 1.0902x over previous
import functools

import jax
import jax.numpy as jnp
from jax import lax
from jax.experimental import pallas as pl
from jax.experimental.pallas import tpu as pltpu
from jax.experimental.pallas import tpu_sc as plsc

N = 10000
E = 160000
NPAD = 10240
NC, NS = 2, 16
NW = NC * NS
B = 128
EPW = 5120
NB = EPW // B
EPAD = NW * EPW
RPT = NPAD // NS
RB = 1000
DC = 128


def _make_sc_scatter(nch, gather=True):
  mesh = plsc.VectorSubcoreMesh(core_axis_name="c", subcore_axis_name="s")

  @functools.partial(
      pl.kernel,
      mesh=mesh,
      out_type=jax.ShapeDtypeStruct((NC, nch, NPAD, DC), jnp.float32),
      scratch_types=[
          pltpu.VMEM((NB, B), jnp.int32),
          pltpu.VMEM((NB, B), jnp.int32),
          pltpu.VMEM((2, B, DC), jnp.float32),
          pltpu.VMEM_SHARED((NPAD, DC), jnp.float32),
          pltpu.SemaphoreType.DMA,
          pltpu.SemaphoreType.DMA,
          pltpu.SemaphoreType.DMA,
          pltpu.SemaphoreType.DMA,
      ],
  )
  def sc_kernel(u_hbm, src_hbm, dst_hbm, zeros_hbm, out_hbm,
                src_v, dst_v, rows_v, acc, g0, g1, s0, s1):
    c = lax.axis_index("c")
    s = lax.axis_index("s")
    wid = c * NS + s
    pltpu.sync_copy(src_hbm.at[wid], src_v)
    pltpu.sync_copy(dst_hbm.at[wid], dst_v)
    gsems = (g0, g1)

    def gth(ch, j, buf, sem):
      return pltpu.async_copy(u_hbm.at[ch].at[src_v.at[j]], rows_v.at[buf],
                              sem)

    def gth_wait(ch, j, buf, sem):
      pltpu.make_async_copy(u_hbm.at[ch].at[src_v.at[j]], rows_v.at[buf],
                            sem).wait()

    def sct(j, buf, sem):
      return pltpu.async_copy(rows_v.at[buf], acc.at[dst_v.at[j]], sem,
                              add=True)

    def sct_wait(j, buf, sem):
      pltpu.make_async_copy(rows_v.at[buf], acc.at[dst_v.at[j]], sem).wait()

    if not gather:
      pltpu.sync_copy(u_hbm, rows_v.at[0])
      pltpu.sync_copy(zeros_hbm, acc.at[pl.ds(s * RPT, RPT)])
      plsc.subcore_barrier()

      def dbody(jj, carry):
        for k in range(8):
          sct(jj * 8 + k, 0, s0)
        for k in range(8):
          sct_wait(jj * 8 + k, 0, s0)
        return carry

      lax.fori_loop(0, NB // 8, dbody, 0)
      plsc.subcore_barrier()
      pltpu.sync_copy(acc.at[pl.ds(s * RPT, RPT)],
                      out_hbm.at[c, 0].at[pl.ds(s * RPT, RPT)])
      return

    for ch in range(nch):
      pltpu.sync_copy(zeros_hbm, acc.at[pl.ds(s * RPT, RPT)])
      gth(ch, 0, 0, g0)
      gth(ch, 1, 1, g1)
      plsc.subcore_barrier()

      def body(jj, carry):
        j0 = 2 * jj
        gth_wait(ch, j0, 0, g0)
        sct(j0, 0, s0)
        gth_wait(ch, j0 + 1, 1, g1)
        sct(j0 + 1, 1, s1)
        sct_wait(j0, 0, s0)
        gth(ch, j0 + 2, 0, g0)
        sct_wait(j0 + 1, 1, s1)
        gth(ch, j0 + 3, 1, g1)
        return carry

      lax.fori_loop(0, NB // 2 - 1, body, 0)
      gth_wait(ch, NB - 2, 0, g0)
      pltpu.sync_copy(rows_v.at[0], acc.at[dst_v.at[NB - 2]], add=True)
      gth_wait(ch, NB - 1, 1, g1)
      pltpu.sync_copy(rows_v.at[1], acc.at[dst_v.at[NB - 1]], add=True)
      plsc.subcore_barrier()
      pltpu.sync_copy(acc.at[pl.ds(s * RPT, RPT)],
                      out_hbm.at[c, ch].at[pl.ds(s * RPT, RPT)])

  return sc_kernel


def _tc_prep_kernel(degp_ref, pos_ref, dinv_ref, u0_ref):
  deg = degp_ref[0, 0, :, 0:1] + degp_ref[1, 0, :, 0:1] + 1.0
  dinv = lax.rsqrt(deg)
  dinv_ref[...] = jnp.broadcast_to(dinv, (RB, 128))
  u0_ref[0] = dinv * pos_ref[...]


def _tc_prep(degp, pos_pad):
  return pl.pallas_call(
      _tc_prep_kernel,
      grid=(N // RB,),
      in_specs=[
          pl.BlockSpec((NC, 1, RB, DC), lambda i: (0, 0, i, 0)),
          pl.BlockSpec((RB, DC), lambda i: (i, 0)),
      ],
      out_specs=[
          pl.BlockSpec((RB, 128), lambda i: (i, 0)),
          pl.BlockSpec((1, RB, DC), lambda i: (0, i, 0)),
      ],
      out_shape=[
          jax.ShapeDtypeStruct((NPAD, 128), jnp.float32),
          jax.ShapeDtypeStruct((1, NPAD, DC), jnp.float32),
      ],
  )(degp, pos_pad)


def _tc_layer_kernel(nch_in, nch_out, d_out,
                     p_ref, u_ref, dinv_ref, w_ref, b_ref, un_ref):
  parts = [p_ref[0, ch] + p_ref[1, ch] + u_ref[ch] for ch in range(nch_in)]
  agg = parts[0] if nch_in == 1 else jnp.concatenate(parts, axis=-1)
  dv = dinv_ref[:, 0:1]
  s = dv * agg
  y = jnp.dot(s, w_ref[...], preferred_element_type=jnp.float32) + b_ref[...]
  x = jnp.where(y >= 0.0, y, 0.01 * y)
  un = dv * x
  if d_out < nch_out * DC:
    un = jnp.concatenate(
        [un, jnp.zeros((RB, nch_out * DC - d_out), jnp.float32)], axis=-1)
  for ch in range(nch_out):
    un_ref[ch] = un[:, ch * DC:(ch + 1) * DC]


def _tc_layer(p, u, dinvb, w, b, nch_in, nch_out):
  d_in, d_out = w.shape
  return pl.pallas_call(
      functools.partial(_tc_layer_kernel, nch_in, nch_out, d_out),
      grid=(N // RB,),
      in_specs=[
          pl.BlockSpec((NC, nch_in, RB, DC), lambda i: (0, 0, i, 0)),
          pl.BlockSpec((nch_in, RB, DC), lambda i: (0, i, 0)),
          pl.BlockSpec((RB, 128), lambda i: (i, 0)),
          pl.BlockSpec((d_in, d_out), lambda i: (0, 0)),
          pl.BlockSpec((1, d_out), lambda i: (0, 0)),
      ],
      out_specs=pl.BlockSpec((nch_out, RB, DC), lambda i: (0, i, 0)),
      out_shape=jax.ShapeDtypeStruct((nch_out, NPAD, DC), jnp.float32),
  )(p, u, dinvb, w, b)


def _tc_final_kernel(nch_in,
                     p_ref, u_ref, dinv_ref, w_ref, b_ref, fcw_ref, fcb_ref,
                     out_ref, macc):
  i = pl.program_id(0)
  parts = [p_ref[0, ch] + p_ref[1, ch] + u_ref[ch] for ch in range(nch_in)]
  agg = jnp.concatenate(parts, axis=-1)
  dv = dinv_ref[:, 0:1]
  s = dv * agg
  y = jnp.dot(s, w_ref[...], preferred_element_type=jnp.float32) + b_ref[...]
  x = jnp.where(y >= 0.0, y, 0.01 * y)
  blk_max = jnp.max(x, axis=0, keepdims=True)

  @pl.when(i == 0)
  def _():
    macc[...] = blk_max

  @pl.when(i > 0)
  def _():
    macc[...] = jnp.maximum(macc[...], blk_max)

  @pl.when(i == pl.num_programs(0) - 1)
  def _():
    r = macc[...]
    out = lax.dot_general(r, fcw_ref[...], (((1,), (1,)), ((), ())),
                          preferred_element_type=jnp.float32)
    out_ref[...] = out + fcb_ref[...]


def _tc_final(p, u, dinvb, w, b, fcw, fcb, nch_in):
  d_in, d_out = w.shape
  return pl.pallas_call(
      functools.partial(_tc_final_kernel, nch_in),
      grid=(N // RB,),
      in_specs=[
          pl.BlockSpec((NC, nch_in, RB, DC), lambda i: (0, 0, i, 0)),
          pl.BlockSpec((nch_in, RB, DC), lambda i: (0, i, 0)),
          pl.BlockSpec((RB, 128), lambda i: (i, 0)),
          pl.BlockSpec((d_in, d_out), lambda i: (0, 0)),
          pl.BlockSpec((1, d_out), lambda i: (0, 0)),
          pl.BlockSpec((1024, 1024), lambda i: (0, 0)),
          pl.BlockSpec((1, 1024), lambda i: (0, 0)),
      ],
      out_specs=pl.BlockSpec((1, 1024), lambda i: (0, 0)),
      out_shape=jax.ShapeDtypeStruct((1, 1024), jnp.float32),
      scratch_shapes=[pltpu.VMEM((1, 1024), jnp.float32)],
  )(p, u, dinvb, w, b, fcw, fcb)


def kernel(pos, edge_index, W1, b1, W2, b2, W3, b3, W4, b4, W5, b5, fcW, fcb):
  f32 = jnp.float32
  src = edge_index[0]
  dst = edge_index[1]
  pad = EPAD - E
  src_r = jnp.concatenate([src, jnp.zeros((pad,), jnp.int32)]).reshape(NW, NB, B)
  dst_r = jnp.concatenate([dst, jnp.full((pad,), N, jnp.int32)]).reshape(NW, NB, B)

  pos_pad = jnp.zeros((NPAD, DC), f32).at[:N, :3].set(pos)
  w1_pad = jnp.zeros((DC, 64), f32).at[:3].set(W1)
  w2_pad = jnp.zeros((DC, 128), f32).at[:64].set(W2)
  zeros_rpt = jnp.zeros((RPT, DC), f32)
  ones_b = jnp.ones((B, DC), f32)

  degp = _make_sc_scatter(1, gather=False)(ones_b, src_r, dst_r, zeros_rpt)

  dinvb, u = _tc_prep(degp, pos_pad)

  layer_ws = [(w1_pad, b1), (w2_pad, b2), (W3, b3), (W4, b4), (W5, b5)]
  nchs = [1, 1, 1, 2, 4]

  for l in range(4):
    w, b = layer_ws[l]
    p = _make_sc_scatter(nchs[l])(u, src_r, dst_r, zeros_rpt)
    u = _tc_layer(p, u, dinvb, w, b.reshape(1, -1), nchs[l], nchs[l + 1])

  w5, b5_ = layer_ws[4]
  p = _make_sc_scatter(nchs[4])(u, src_r, dst_r, zeros_rpt)
  out = _tc_final(p, u, dinvb, w5, b5_.reshape(1, -1), fcW, fcb.reshape(1, -1),
                  nchs[4])
  return out.reshape(-1)


# speedup vs baseline: 5.6483x; 1.0461x over previous
import functools

import jax
import jax.numpy as jnp
from jax import lax
from jax.experimental import pallas as pl
from jax.experimental.pallas import tpu as pltpu
from jax.experimental.pallas import tpu_sc as plsc

N = 10000
E = 160000
NPAD = 10240
NC, NS = 2, 16
NW = NC * NS
B = 128
EPW = 5120
NB = EPW // B
EPAD = NW * EPW
RPT = NPAD // NS
RB = 1000
DC = 128


def _make_sc_scatter(nch):
  mesh = plsc.VectorSubcoreMesh(core_axis_name="c", subcore_axis_name="s")

  @functools.partial(
      pl.kernel,
      mesh=mesh,
      out_type=jax.ShapeDtypeStruct((NC, nch, NPAD, DC), jnp.float32),
      scratch_types=[
          pltpu.VMEM((NB, B), jnp.int32),
          pltpu.VMEM((NB, B), jnp.int32),
          pltpu.VMEM((2, B, DC), jnp.float32),
          pltpu.VMEM_SHARED((NPAD, DC), jnp.float32),
          pltpu.SemaphoreType.DMA,
          pltpu.SemaphoreType.DMA,
          pltpu.SemaphoreType.DMA,
          pltpu.SemaphoreType.DMA,
      ],
  )
  def sc_kernel(u_hbm, src_hbm, dst_hbm, zeros_hbm, out_hbm,
                src_v, dst_v, rows_v, acc, g0, g1, s0, s1):
    c = lax.axis_index("c")
    s = lax.axis_index("s")
    wid = c * NS + s
    pltpu.sync_copy(src_hbm.at[wid], src_v)
    pltpu.sync_copy(dst_hbm.at[wid], dst_v)
    gs = (g0, g1)
    ss = (s0, s1)

    def gth(ch, j, buf, sem):
      return pltpu.async_copy(u_hbm.at[ch].at[src_v.at[j]], rows_v.at[buf],
                              sem)

    def gth_wait(ch, j, buf, sem):
      pltpu.make_async_copy(u_hbm.at[ch].at[src_v.at[j]], rows_v.at[buf],
                            sem).wait()

    def sct(j, buf, sem):
      return pltpu.async_copy(rows_v.at[buf], acc.at[dst_v.at[j]], sem,
                              add=True)

    def sct_wait(j, buf, sem):
      pltpu.make_async_copy(rows_v.at[buf], acc.at[dst_v.at[j]], sem).wait()

    for b in range(2):
      gth(0, b, b, gs[b])
    for ch in range(nch):
      pltpu.sync_copy(zeros_hbm, acc.at[pl.ds(s * RPT, RPT)])
      plsc.subcore_barrier()

      def body(jj, carry):
        j0 = 2 * jj
        for b in range(2):
          gth_wait(ch, j0 + b, b, gs[b])
          sct(j0 + b, b, ss[b])
        for b in range(2):
          sct_wait(j0 + b, b, ss[b])
          gth(ch, j0 + 2 + b, b, gs[b])
        return carry

      lax.fori_loop(0, NB // 2 - 1, body, 0)
      for b in range(2):
        gth_wait(ch, NB - 2 + b, b, gs[b])
        sct(NB - 2 + b, b, ss[b])
      for b in range(2):
        sct_wait(NB - 2 + b, b, ss[b])
        if ch + 1 < nch:
          gth(ch + 1, b, b, gs[b])
      plsc.subcore_barrier()
      pltpu.sync_copy(acc.at[pl.ds(s * RPT, RPT)],
                      out_hbm.at[c, ch].at[pl.ds(s * RPT, RPT)])

  return sc_kernel


def _make_sc_degree():
  mesh = plsc.VectorSubcoreMesh(core_axis_name="c", subcore_axis_name="s")

  @functools.partial(
      pl.kernel,
      mesh=mesh,
      out_type=jax.ShapeDtypeStruct((NC, 1, NPAD, DC), jnp.float32),
      scratch_types=[
          pltpu.VMEM((NB, B), jnp.int32),
          pltpu.VMEM((B, DC), jnp.float32),
          pltpu.VMEM_SHARED((NPAD, DC), jnp.float32),
          pltpu.SemaphoreType.DMA,
      ],
  )
  def sc_kernel(ones_hbm, dst_hbm, zeros_hbm, out_hbm,
                dst_v, ones_v, acc, s0):
    c = lax.axis_index("c")
    s = lax.axis_index("s")
    wid = c * NS + s
    pltpu.sync_copy(dst_hbm.at[wid], dst_v)
    pltpu.sync_copy(ones_hbm, ones_v)
    pltpu.sync_copy(zeros_hbm, acc.at[pl.ds(s * RPT, RPT)])
    plsc.subcore_barrier()

    def dbody(jj, carry):
      for k in range(8):
        pltpu.async_copy(ones_v, acc.at[dst_v.at[jj * 8 + k]], s0, add=True)
      for k in range(8):
        pltpu.make_async_copy(ones_v, acc.at[dst_v.at[jj * 8 + k]], s0).wait()
      return carry

    lax.fori_loop(0, NB // 8, dbody, 0)
    plsc.subcore_barrier()
    pltpu.sync_copy(acc.at[pl.ds(s * RPT, RPT)],
                    out_hbm.at[c, 0].at[pl.ds(s * RPT, RPT)])

  return sc_kernel


def _tc_prep_kernel(degp_ref, pos_ref, dinv_ref, u0_ref):
  deg = degp_ref[0, 0, :, 0:1] + degp_ref[1, 0, :, 0:1] + 1.0
  dinv = lax.rsqrt(deg)
  dinv_ref[...] = jnp.broadcast_to(dinv, (RB, 128))
  u0_ref[0] = dinv * pos_ref[...]


def _tc_prep(degp, pos_pad):
  return pl.pallas_call(
      _tc_prep_kernel,
      grid=(N // RB,),
      in_specs=[
          pl.BlockSpec((NC, 1, RB, DC), lambda i: (0, 0, i, 0)),
          pl.BlockSpec((RB, DC), lambda i: (i, 0)),
      ],
      out_specs=[
          pl.BlockSpec((RB, 128), lambda i: (i, 0)),
          pl.BlockSpec((1, RB, DC), lambda i: (0, i, 0)),
      ],
      out_shape=[
          jax.ShapeDtypeStruct((NPAD, 128), jnp.float32),
          jax.ShapeDtypeStruct((1, NPAD, DC), jnp.float32),
      ],
  )(degp, pos_pad)


def _tc_layer_kernel(nch_in, nch_out, d_out,
                     p_ref, u_ref, dinv_ref, w_ref, b_ref, un_ref):
  parts = [p_ref[0, ch] + p_ref[1, ch] + u_ref[ch] for ch in range(nch_in)]
  agg = parts[0] if nch_in == 1 else jnp.concatenate(parts, axis=-1)
  dv = dinv_ref[:, 0:1]
  s = dv * agg
  y = jnp.dot(s, w_ref[...], preferred_element_type=jnp.float32) + b_ref[...]
  x = jnp.where(y >= 0.0, y, 0.01 * y)
  un = dv * x
  if d_out < nch_out * DC:
    un = jnp.concatenate(
        [un, jnp.zeros((RB, nch_out * DC - d_out), jnp.float32)], axis=-1)
  for ch in range(nch_out):
    un_ref[ch] = un[:, ch * DC:(ch + 1) * DC]


def _tc_layer(p, u, dinvb, w, b, nch_in, nch_out):
  d_in, d_out = w.shape
  return pl.pallas_call(
      functools.partial(_tc_layer_kernel, nch_in, nch_out, d_out),
      grid=(N // RB,),
      in_specs=[
          pl.BlockSpec((NC, nch_in, RB, DC), lambda i: (0, 0, i, 0)),
          pl.BlockSpec((nch_in, RB, DC), lambda i: (0, i, 0)),
          pl.BlockSpec((RB, 128), lambda i: (i, 0)),
          pl.BlockSpec((d_in, d_out), lambda i: (0, 0)),
          pl.BlockSpec((1, d_out), lambda i: (0, 0)),
      ],
      out_specs=pl.BlockSpec((nch_out, RB, DC), lambda i: (0, i, 0)),
      out_shape=jax.ShapeDtypeStruct((nch_out, NPAD, DC), jnp.float32),
  )(p, u, dinvb, w, b)


def _tc_final_kernel(nch_in,
                     p_ref, u_ref, dinv_ref, w_ref, b_ref, fcw_ref, fcb_ref,
                     out_ref, macc):
  i = pl.program_id(0)
  parts = [p_ref[0, ch] + p_ref[1, ch] + u_ref[ch] for ch in range(nch_in)]
  agg = jnp.concatenate(parts, axis=-1)
  dv = dinv_ref[:, 0:1]
  s = dv * agg
  y = jnp.dot(s, w_ref[...], preferred_element_type=jnp.float32) + b_ref[...]
  x = jnp.where(y >= 0.0, y, 0.01 * y)
  blk_max = jnp.max(x, axis=0, keepdims=True)

  @pl.when(i == 0)
  def _():
    macc[...] = blk_max

  @pl.when(i > 0)
  def _():
    macc[...] = jnp.maximum(macc[...], blk_max)

  @pl.when(i == pl.num_programs(0) - 1)
  def _():
    r = macc[...]
    out = lax.dot_general(r, fcw_ref[...], (((1,), (1,)), ((), ())),
                          preferred_element_type=jnp.float32)
    out_ref[...] = out + fcb_ref[...]


def _tc_final(p, u, dinvb, w, b, fcw, fcb, nch_in):
  d_in, d_out = w.shape
  return pl.pallas_call(
      functools.partial(_tc_final_kernel, nch_in),
      grid=(N // RB,),
      in_specs=[
          pl.BlockSpec((NC, nch_in, RB, DC), lambda i: (0, 0, i, 0)),
          pl.BlockSpec((nch_in, RB, DC), lambda i: (0, i, 0)),
          pl.BlockSpec((RB, 128), lambda i: (i, 0)),
          pl.BlockSpec((d_in, d_out), lambda i: (0, 0)),
          pl.BlockSpec((1, d_out), lambda i: (0, 0)),
          pl.BlockSpec((1024, 1024), lambda i: (0, 0)),
          pl.BlockSpec((1, 1024), lambda i: (0, 0)),
      ],
      out_specs=pl.BlockSpec((1, 1024), lambda i: (0, 0)),
      out_shape=jax.ShapeDtypeStruct((1, 1024), jnp.float32),
      scratch_shapes=[pltpu.VMEM((1, 1024), jnp.float32)],
  )(p, u, dinvb, w, b, fcw, fcb)


def kernel(pos, edge_index, W1, b1, W2, b2, W3, b3, W4, b4, W5, b5, fcW, fcb):
  f32 = jnp.float32
  src = edge_index[0]
  dst = edge_index[1]
  pad = EPAD - E
  src_r = jnp.concatenate([src, jnp.zeros((pad,), jnp.int32)]).reshape(NW, NB, B)
  dst_r = jnp.concatenate([dst, jnp.full((pad,), N, jnp.int32)]).reshape(NW, NB, B)

  pos_pad = jnp.zeros((NPAD, DC), f32).at[:N, :3].set(pos)
  w1_pad = jnp.zeros((DC, 64), f32).at[:3].set(W1)
  w2_pad = jnp.zeros((DC, 128), f32).at[:64].set(W2)
  zeros_rpt = jnp.zeros((RPT, DC), f32)
  ones_b = jnp.ones((B, DC), f32)

  degp = _make_sc_degree()(ones_b, dst_r, zeros_rpt)

  dinvb, u = _tc_prep(degp, pos_pad)

  layer_ws = [(w1_pad, b1), (w2_pad, b2), (W3, b3), (W4, b4), (W5, b5)]
  nchs = [1, 1, 1, 2, 4]

  for l in range(4):
    w, b = layer_ws[l]
    p = _make_sc_scatter(nchs[l])(u, src_r, dst_r, zeros_rpt)
    u = _tc_layer(p, u, dinvb, w, b.reshape(1, -1), nchs[l], nchs[l + 1])

  w5, b5_ = layer_ws[4]
  p = _make_sc_scatter(nchs[4])(u, src_r, dst_r, zeros_rpt)
  out = _tc_final(p, u, dinvb, w5, b5_.reshape(1, -1), fcW, fcb.reshape(1, -1),
                  nchs[4])
  return out.reshape(-1)


# speedup vs baseline: 14.6824x; 2.5994x over previous
import functools

import jax
import jax.numpy as jnp
from jax import lax
from jax.experimental import pallas as pl
from jax.experimental.pallas import tpu as pltpu
from jax.experimental.pallas import tpu_sc as plsc

N = 10000
E = 160000
NPAD = 10240
NC, NS = 2, 16
NW = NC * NS
B = 128
EPW = 5120
NB = EPW // B
EPAD = NW * EPW
RPT = NPAD // NS
RB = 1000
DC = 128


def _make_sc_scatter(nch):
  mesh = plsc.VectorSubcoreMesh(core_axis_name="c", subcore_axis_name="s")

  @functools.partial(
      pl.kernel,
      mesh=mesh,
      out_type=jax.ShapeDtypeStruct((NC, nch, NPAD, DC), jnp.float32),
      scratch_types=[
          pltpu.VMEM((NB, B), jnp.int32),
          pltpu.VMEM((NB, B), jnp.int32),
          pltpu.VMEM((2, B, DC), jnp.float32),
          pltpu.VMEM_SHARED((NPAD, DC), jnp.float32),
          pltpu.SemaphoreType.DMA,
          pltpu.SemaphoreType.DMA,
          pltpu.SemaphoreType.DMA,
          pltpu.SemaphoreType.DMA,
      ],
  )
  def sc_kernel(u_hbm, src_hbm, dst_hbm, zeros_hbm, out_hbm,
                src_v, dst_v, rows_v, acc, g0, g1, s0, s1):
    c = lax.axis_index("c")
    s = lax.axis_index("s")
    wid = c * NS + s
    pltpu.sync_copy(src_hbm.at[wid], src_v)
    pltpu.sync_copy(dst_hbm.at[wid], dst_v)
    gs = (g0, g1)
    ss = (s0, s1)

    def gth(ch, j, buf, sem):
      return pltpu.async_copy(u_hbm.at[ch].at[src_v.at[j]], rows_v.at[buf],
                              sem)

    def gth_wait(ch, j, buf, sem):
      pltpu.make_async_copy(u_hbm.at[ch].at[src_v.at[j]], rows_v.at[buf],
                            sem).wait()

    def sct(j, buf, sem):
      return pltpu.async_copy(rows_v.at[buf], acc.at[dst_v.at[j]], sem,
                              add=True)

    def sct_wait(j, buf, sem):
      pltpu.make_async_copy(rows_v.at[buf], acc.at[dst_v.at[j]], sem).wait()

    for b in range(2):
      gth(0, b, b, gs[b])
    for ch in range(nch):
      pltpu.sync_copy(zeros_hbm, acc.at[pl.ds(s * RPT, RPT)])
      plsc.subcore_barrier()

      def body(jj, carry):
        j0 = 2 * jj
        for b in range(2):
          gth_wait(ch, j0 + b, b, gs[b])
          sct(j0 + b, b, ss[b])
        for b in range(2):
          sct_wait(j0 + b, b, ss[b])
          gth(ch, j0 + 2 + b, b, gs[b])
        return carry

      lax.fori_loop(0, NB // 2 - 1, body, 0)
      for b in range(2):
        gth_wait(ch, NB - 2 + b, b, gs[b])
        sct(NB - 2 + b, b, ss[b])
      for b in range(2):
        sct_wait(NB - 2 + b, b, ss[b])
        if ch + 1 < nch:
          gth(ch + 1, b, b, gs[b])
      plsc.subcore_barrier()
      pltpu.sync_copy(acc.at[pl.ds(s * RPT, RPT)],
                      out_hbm.at[c, ch].at[pl.ds(s * RPT, RPT)])

  return sc_kernel


def _make_sc_degree():
  mesh = plsc.VectorSubcoreMesh(core_axis_name="c", subcore_axis_name="s")

  @functools.partial(
      pl.kernel,
      mesh=mesh,
      out_type=jax.ShapeDtypeStruct((NC, 1, NPAD, DC), jnp.float32),
      scratch_types=[
          pltpu.VMEM((NB, B), jnp.int32),
          pltpu.VMEM((B, DC), jnp.float32),
          pltpu.VMEM_SHARED((NPAD, DC), jnp.float32),
          pltpu.SemaphoreType.DMA,
      ],
  )
  def sc_kernel(ones_hbm, dst_hbm, zeros_hbm, out_hbm,
                dst_v, ones_v, acc, s0):
    c = lax.axis_index("c")
    s = lax.axis_index("s")
    wid = c * NS + s
    pltpu.sync_copy(dst_hbm.at[wid], dst_v)
    pltpu.sync_copy(ones_hbm, ones_v)
    pltpu.sync_copy(zeros_hbm, acc.at[pl.ds(s * RPT, RPT)])
    plsc.subcore_barrier()

    def dbody(jj, carry):
      for k in range(8):
        pltpu.async_copy(ones_v, acc.at[dst_v.at[jj * 8 + k]], s0, add=True)
      for k in range(8):
        pltpu.make_async_copy(ones_v, acc.at[dst_v.at[jj * 8 + k]], s0).wait()
      return carry

    lax.fori_loop(0, NB // 8, dbody, 0)
    plsc.subcore_barrier()
    pltpu.sync_copy(acc.at[pl.ds(s * RPT, RPT)],
                    out_hbm.at[c, 0].at[pl.ds(s * RPT, RPT)])

  return sc_kernel


def _tc_prep_kernel(degp_ref, pos_ref, dinv_ref, u0_ref):
  deg = degp_ref[0, 0, :, 0:1] + degp_ref[1, 0, :, 0:1] + 1.0
  dinv = lax.rsqrt(deg)
  dinv_ref[...] = jnp.broadcast_to(dinv, (RB, 128))
  u0_ref[0] = dinv * pos_ref[...]


def _tc_prep(degp, pos_pad):
  return pl.pallas_call(
      _tc_prep_kernel,
      grid=(N // RB,),
      in_specs=[
          pl.BlockSpec((NC, 1, RB, DC), lambda i: (0, 0, i, 0)),
          pl.BlockSpec((RB, DC), lambda i: (i, 0)),
      ],
      out_specs=[
          pl.BlockSpec((RB, 128), lambda i: (i, 0)),
          pl.BlockSpec((1, RB, DC), lambda i: (0, i, 0)),
      ],
      out_shape=[
          jax.ShapeDtypeStruct((NPAD, 128), jnp.float32),
          jax.ShapeDtypeStruct((1, NPAD, DC), jnp.float32),
      ],
  )(degp, pos_pad)


def _tc_layer_kernel(nch_in, nch_out, d_out,
                     p_ref, u_ref, dinv_ref, w_ref, b_ref, un_ref):
  parts = [p_ref[0, ch] + p_ref[1, ch] + u_ref[ch] for ch in range(nch_in)]
  agg = parts[0] if nch_in == 1 else jnp.concatenate(parts, axis=-1)
  dv = dinv_ref[:, 0:1]
  s = dv * agg
  y = jnp.dot(s, w_ref[...], preferred_element_type=jnp.float32) + b_ref[...]
  x = jnp.where(y >= 0.0, y, 0.01 * y)
  un = dv * x
  if d_out < nch_out * DC:
    un = jnp.concatenate(
        [un, jnp.zeros((RB, nch_out * DC - d_out), jnp.float32)], axis=-1)
  for ch in range(nch_out):
    un_ref[ch] = un[:, ch * DC:(ch + 1) * DC]


def _tc_layer(p, u, dinvb, w, b, nch_in, nch_out):
  d_in, d_out = w.shape
  return pl.pallas_call(
      functools.partial(_tc_layer_kernel, nch_in, nch_out, d_out),
      grid=(N // RB,),
      in_specs=[
          pl.BlockSpec((NC, nch_in, RB, DC), lambda i: (0, 0, i, 0)),
          pl.BlockSpec((nch_in, RB, DC), lambda i: (0, i, 0)),
          pl.BlockSpec((RB, 128), lambda i: (i, 0)),
          pl.BlockSpec((d_in, d_out), lambda i: (0, 0)),
          pl.BlockSpec((1, d_out), lambda i: (0, 0)),
      ],
      out_specs=pl.BlockSpec((nch_out, RB, DC), lambda i: (0, i, 0)),
      out_shape=jax.ShapeDtypeStruct((nch_out, NPAD, DC), jnp.float32),
  )(p, u, dinvb, w, b)


def _tc_final_kernel(nch_in,
                     p_ref, u_ref, dinv_ref, w_ref, b_ref, fcw_ref, fcb_ref,
                     out_ref, macc):
  i = pl.program_id(0)
  parts = [p_ref[0, ch] + p_ref[1, ch] + u_ref[ch] for ch in range(nch_in)]
  agg = jnp.concatenate(parts, axis=-1)
  dv = dinv_ref[:, 0:1]
  s = dv * agg
  y = jnp.dot(s, w_ref[...], preferred_element_type=jnp.float32) + b_ref[...]
  x = jnp.where(y >= 0.0, y, 0.01 * y)
  blk_max = jnp.max(x, axis=0, keepdims=True)

  @pl.when(i == 0)
  def _():
    macc[...] = blk_max

  @pl.when(i > 0)
  def _():
    macc[...] = jnp.maximum(macc[...], blk_max)

  @pl.when(i == pl.num_programs(0) - 1)
  def _():
    r = macc[...]
    out = lax.dot_general(r, fcw_ref[...], (((1,), (1,)), ((), ())),
                          preferred_element_type=jnp.float32)
    out_ref[...] = out + fcb_ref[...]


def _tc_final(p, u, dinvb, w, b, fcw, fcb, nch_in):
  d_in, d_out = w.shape
  return pl.pallas_call(
      functools.partial(_tc_final_kernel, nch_in),
      grid=(N // RB,),
      in_specs=[
          pl.BlockSpec((NC, nch_in, RB, DC), lambda i: (0, 0, i, 0)),
          pl.BlockSpec((nch_in, RB, DC), lambda i: (0, i, 0)),
          pl.BlockSpec((RB, 128), lambda i: (i, 0)),
          pl.BlockSpec((d_in, d_out), lambda i: (0, 0)),
          pl.BlockSpec((1, d_out), lambda i: (0, 0)),
          pl.BlockSpec((1024, 1024), lambda i: (0, 0)),
          pl.BlockSpec((1, 1024), lambda i: (0, 0)),
      ],
      out_specs=pl.BlockSpec((1, 1024), lambda i: (0, 0)),
      out_shape=jax.ShapeDtypeStruct((1, 1024), jnp.float32),
      scratch_shapes=[pltpu.VMEM((1, 1024), jnp.float32)],
  )(p, u, dinvb, w, b, fcw, fcb)


def kernel(pos, edge_index, W1, b1, W2, b2, W3, b3, W4, b4, W5, b5, fcW, fcb):
  f32 = jnp.float32
  src = edge_index[0]
  dst = edge_index[1]
  pad = EPAD - E
  pidx = jnp.arange(pad, dtype=jnp.int32)
  src_r = jnp.concatenate([src, pidx % N]).reshape(NW, NB, B)
  dst_r = jnp.concatenate([dst, N + pidx % (NPAD - N)]).reshape(NW, NB, B)

  pos_pad = jnp.zeros((NPAD, DC), f32).at[:N, :3].set(pos)
  w1_pad = jnp.zeros((DC, 64), f32).at[:3].set(W1)
  w2_pad = jnp.zeros((DC, 128), f32).at[:64].set(W2)
  zeros_rpt = jnp.zeros((RPT, DC), f32)
  ones_b = jnp.ones((B, DC), f32)

  degp = _make_sc_degree()(ones_b, dst_r, zeros_rpt)

  dinvb, u = _tc_prep(degp, pos_pad)

  layer_ws = [(w1_pad, b1), (w2_pad, b2), (W3, b3), (W4, b4), (W5, b5)]
  nchs = [1, 1, 1, 2, 4]

  for l in range(4):
    w, b = layer_ws[l]
    p = _make_sc_scatter(nchs[l])(u, src_r, dst_r, zeros_rpt)
    u = _tc_layer(p, u, dinvb, w, b.reshape(1, -1), nchs[l], nchs[l + 1])

  w5, b5_ = layer_ws[4]
  p = _make_sc_scatter(nchs[4])(u, src_r, dst_r, zeros_rpt)
  out = _tc_final(p, u, dinvb, w5, b5_.reshape(1, -1), fcW, fcb.reshape(1, -1),
                  nchs[4])
  return out.reshape(-1)
